# baseline (speedup 1.0000x reference)
import functools

import jax
import jax.numpy as jnp
from jax import lax
from jax.experimental import pallas as pl
from jax.experimental.pallas import tpu as pltpu
from jax.experimental.pallas import tpu_sc as plsc

N = 3200000
NUM_ROWS = 100000
NT = 16
PER_T = N // NT
G = 10000
NG = PER_T // G
L = 16

_mesh = plsc.VectorSubcoreMesh(core_axis_name="c", subcore_axis_name="s")


@functools.partial(
    pl.kernel,
    mesh=_mesh,
    compiler_params=pltpu.CompilerParams(needs_layout_passes=False),
    out_type=(
        jax.ShapeDtypeStruct((N,), jnp.int32),
        jax.ShapeDtypeStruct((N,), jnp.int32),
    ),
    scratch_types=[
        pltpu.VMEM((NUM_ROWS,), jnp.int32),
        pltpu.VMEM((G,), jnp.uint32),
        pltpu.VMEM((G,), jnp.int32),
        pltpu.SemaphoreType.DMA,
    ],
)
def _sc_gather(idx_hbm, lo_hbm, hi_hbm, out_lo_hbm, out_hi_hbm,
               plane_v, idx_v, vals_v, sem):
    wid = lax.axis_index("s") * 2 + lax.axis_index("c")
    is_lo = wid < jnp.int32(NT)
    slot = lax.rem(wid, jnp.int32(NT))
    base = slot * jnp.int32(PER_T)

    @pl.when(is_lo)
    def _():
        pltpu.sync_copy(lo_hbm, plane_v)

    @pl.when(jnp.logical_not(is_lo))
    def _():
        pltpu.sync_copy(hi_hbm, plane_v)

    def group(g, carry):
        off = base + g * jnp.int32(G)
        pltpu.sync_copy(idx_hbm.at[pl.ds(off, G)], idx_v)

        @plsc.parallel_loop(jnp.int32(0), jnp.int32(G), step=jnp.int32(L),
                            unroll=8)
        def gbody(i):
            ids = plsc.bitcast(idx_v[pl.ds(i, L)], jnp.int32)
            vals_v[pl.ds(i, L)] = plsc.load_gather(plane_v, [ids])

        @pl.when(is_lo)
        def _():
            pltpu.sync_copy(vals_v, out_lo_hbm.at[pl.ds(off, G)])

        @pl.when(jnp.logical_not(is_lo))
        def _():
            pltpu.sync_copy(vals_v, out_hi_hbm.at[pl.ds(off, G)])

        return carry

    lax.fori_loop(jnp.int32(0), jnp.int32(NG), group, 0)


def kernel(n_id, last_update):
    idx32 = n_id.astype(jnp.uint32)
    table_lo = last_update.astype(jnp.int32)
    table_hi = (last_update >> 32).astype(jnp.int32)
    out_lo, out_hi = _sc_gather(idx32, table_lo, table_hi)
    return (out_hi.astype(jnp.int64) << 32) | (
        out_lo.astype(jnp.uint32).astype(jnp.int64))


# speedup vs baseline: 1.0016x; 1.0016x over previous
import functools

import jax
import jax.numpy as jnp
from jax import lax
from jax.experimental import pallas as pl
from jax.experimental.pallas import tpu as pltpu
from jax.experimental.pallas import tpu_sc as plsc

N = 3200000
NUM_ROWS = 100000
NT = 16
PER_T = N // NT
G = 10000
NG = PER_T // G
L = 16

_mesh = plsc.VectorSubcoreMesh(core_axis_name="c", subcore_axis_name="s")


@functools.partial(
    pl.kernel,
    mesh=_mesh,
    compiler_params=pltpu.CompilerParams(needs_layout_passes=False),
    out_type=(
        jax.ShapeDtypeStruct((N,), jnp.int32),
        jax.ShapeDtypeStruct((N,), jnp.int32),
    ),
    scratch_types=[
        pltpu.VMEM((NUM_ROWS,), jnp.int32),
        pltpu.VMEM((G,), jnp.uint32),
        pltpu.VMEM((G,), jnp.int32),
        pltpu.SemaphoreType.DMA,
    ],
)
def _sc_gather(idx_hbm, lo_hbm, hi_hbm, out_lo_hbm, out_hi_hbm,
               plane_v, idx_v, vals_v, sem):
    wid = lax.axis_index("s") * 2 + lax.axis_index("c")
    is_lo = wid < jnp.int32(NT)
    slot = lax.rem(wid, jnp.int32(NT))
    base = slot * jnp.int32(PER_T)

    @pl.when(is_lo)
    def _():
        pltpu.sync_copy(lo_hbm, plane_v)

    @pl.when(jnp.logical_not(is_lo))
    def _():
        pltpu.sync_copy(hi_hbm, plane_v)

    def group(g, carry):
        off = base + g * jnp.int32(G)
        pltpu.sync_copy(idx_hbm.at[pl.ds(off, G)], idx_v)

        @plsc.parallel_loop(jnp.int32(0), jnp.int32(G), step=jnp.int32(L),
                            unroll=16)
        def gbody(i):
            ids = plsc.bitcast(idx_v[pl.ds(i, L)], jnp.int32)
            vals_v[pl.ds(i, L)] = plsc.load_gather(plane_v, [ids])

        @pl.when(is_lo)
        def _():
            pltpu.sync_copy(vals_v, out_lo_hbm.at[pl.ds(off, G)])

        @pl.when(jnp.logical_not(is_lo))
        def _():
            pltpu.sync_copy(vals_v, out_hi_hbm.at[pl.ds(off, G)])

        return carry

    lax.fori_loop(jnp.int32(0), jnp.int32(NG), group, 0)


def kernel(n_id, last_update):
    idx32 = n_id.astype(jnp.uint32)
    table_lo = last_update.astype(jnp.int32)
    table_hi = (last_update >> 32).astype(jnp.int32)
    out_lo, out_hi = _sc_gather(idx32, table_lo, table_hi)
    return (out_hi.astype(jnp.int64) << 32) | (
        out_lo.astype(jnp.uint32).astype(jnp.int64))


# speedup vs baseline: 1.0017x; 1.0002x over previous
import functools

import jax
import jax.numpy as jnp
from jax import lax
from jax.experimental import pallas as pl
from jax.experimental.pallas import tpu as pltpu
from jax.experimental.pallas import tpu_sc as plsc

N = 3200000
NUM_ROWS = 100000
NT = 16
PER_T = N // NT
G = 10000
NG = PER_T // G
L = 16

_mesh = plsc.VectorSubcoreMesh(core_axis_name="c", subcore_axis_name="s")


@functools.partial(
    pl.kernel,
    mesh=_mesh,
    compiler_params=pltpu.CompilerParams(needs_layout_passes=False),
    out_type=(
        jax.ShapeDtypeStruct((N,), jnp.uint32),
        jax.ShapeDtypeStruct((N,), jnp.uint32),
    ),
    scratch_types=[
        pltpu.VMEM((NUM_ROWS,), jnp.int32),
        pltpu.VMEM((G,), jnp.uint32),
        pltpu.VMEM((G,), jnp.int32),
        pltpu.SemaphoreType.DMA,
    ],
)
def _sc_gather(idx_hbm, lo_hbm, hi_hbm, out_lo_hbm, out_hi_hbm,
               plane_v, idx_v, vals_v, sem):
    wid = lax.axis_index("s") * 2 + lax.axis_index("c")
    is_lo = wid < jnp.int32(NT)
    slot = lax.rem(wid, jnp.int32(NT))
    base = slot * jnp.int32(PER_T)

    @pl.when(is_lo)
    def _():
        pltpu.sync_copy(lo_hbm, plane_v)

    @pl.when(jnp.logical_not(is_lo))
    def _():
        pltpu.sync_copy(hi_hbm, plane_v)

    def group(g, carry):
        off = base + g * jnp.int32(G)
        pltpu.sync_copy(idx_hbm.at[pl.ds(off, G)], idx_v)

        @plsc.parallel_loop(jnp.int32(0), jnp.int32(G), step=jnp.int32(L),
                            unroll=16)
        def gbody(i):
            ids = plsc.bitcast(idx_v[pl.ds(i, L)], jnp.int32)
            vals_v[pl.ds(i, L)] = plsc.load_gather(plane_v, [ids])

        @pl.when(is_lo)
        def _():
            pltpu.sync_copy(vals_v, out_lo_hbm.at[pl.ds(off, G)])

        @pl.when(jnp.logical_not(is_lo))
        def _():
            pltpu.sync_copy(vals_v, out_hi_hbm.at[pl.ds(off, G)])

        return carry

    lax.fori_loop(jnp.int32(0), jnp.int32(NG), group, 0)


def kernel(n_id, last_update):
    idx32 = n_id.astype(jnp.uint32)
    table_lo = last_update.astype(jnp.int32)
    table_hi = (last_update >> 32).astype(jnp.int32)
    out_lo, out_hi = _sc_gather(idx32, table_lo, table_hi)
    return (out_hi.astype(jnp.int64) << 32) | out_lo.astype(jnp.int64)
